# baseline (speedup 1.0000x reference)
import functools
import math

import jax
import jax.numpy as jnp
from jax import lax
from jax.experimental import pallas as pl
from jax.experimental.pallas import tpu as pltpu
from jax.experimental.pallas import tpu_sc as plsc

N = 10000
E = 320000
IN = 128
HID = 32
H = 4
OUT = 128
DQK = H * HID

NC = 2
NS = 16
NW = NC * NS
L = 16

CHUNK = 64
IB = 4
CH_PER_W = 160
NBLK = CH_PER_W // IB
EW = CHUNK * CH_PER_W
EPAD = EW * NW
NPAD = 10112
ROWS_PER_TILE = NPAD // NS
WACC = 144
INV_SQRT_HID = 1.0 / math.sqrt(HID)


def _proj_body(x_ref, wq_ref, wk_ref, q_ref, k_ref):
    xb = x_ref[...]
    dn = (((1,), (1,)), ((), ()))
    q = lax.dot_general(xb, wq_ref[...], dn, preferred_element_type=jnp.float32)
    k = lax.dot_general(xb, wk_ref[...], dn, preferred_element_type=jnp.float32)
    qq = jnp.concatenate(
        [q, jnp.ones((N, H), jnp.float32), jnp.zeros((N, WACC - DQK - H), jnp.float32)],
        axis=1)
    q_ref[pl.ds(0, N), :] = qq
    k_ref[pl.ds(0, N), :] = k
    q_ref[pl.ds(N, NPAD - N), :] = jnp.zeros((NPAD - N, WACC), jnp.float32)
    k_ref[pl.ds(N, NPAD - N), :] = jnp.zeros((NPAD - N, DQK), jnp.float32)


def _project(x, wq2, wk2):
    return pl.pallas_call(
        _proj_body,
        out_shape=[jax.ShapeDtypeStruct((NPAD, WACC), jnp.float32),
                   jax.ShapeDtypeStruct((NPAD, DQK), jnp.float32)],
    )(x, wq2, wk2)


def _edge_body(q_hbm, k_hbm, row_hbm, col_hbm, out_hbm,
               rowA, colA, rowB, colB, qb0, kb0, qb1, kb1, acc_sh,
               sem_iA, sem_iB, sem_g0, sem_g1):
    c = lax.axis_index("c")
    s = lax.axis_index("s")
    wid = s * NC + c

    zeros16 = jnp.zeros((L,), jnp.float32)
    iota16 = lax.iota(jnp.int32, L)
    qbs = (qb0, qb1)
    kbs = (kb0, kb1)
    blks = ((rowA, colA, sem_iA), (rowB, colB, sem_iB))

    def _zrow(i, carry):
        ri = jnp.full((L,), i, jnp.int32)
        for j in range(WACC // L):
            plsc.store_scatter(qb0, [ri, iota16 + j * L], zeros16)
        return carry
    lax.fori_loop(0, CHUNK, _zrow, 0)
    base_r = s * ROWS_PER_TILE
    for t in range(ROWS_PER_TILE // CHUNK):
        pltpu.sync_copy(qb0, acc_sh.at[pl.ds(base_r + t * CHUNK, CHUNK)])
    rem = ROWS_PER_TILE % CHUNK
    if rem:
        pltpu.sync_copy(qb0.at[pl.ds(0, rem)],
                        acc_sh.at[pl.ds(base_r + (ROWS_PER_TILE // CHUNK) * CHUNK, rem)])
    plsc.subcore_barrier()

    cbase = wid * CH_PER_W

    pltpu.sync_copy(row_hbm.at[pl.ds(cbase, IB)], rowA)
    pltpu.sync_copy(col_hbm.at[pl.ds(cbase, IB)], colA)
    pltpu.async_copy(row_hbm.at[pl.ds(cbase + IB, IB)], rowB, sem_iB)
    pltpu.async_copy(col_hbm.at[pl.ds(cbase + IB, IB)], colB, sem_iB)
    pltpu.async_copy(q_hbm.at[rowA.at[0]], qb0, sem_g0)
    pltpu.async_copy(k_hbm.at[colA.at[0]], kb0, sem_g0)

    def _compute(qb, kb):
        def _gh(i, carry):
            g = i // H
            h = i - g * H
            lanes = iota16 + g * L
            hbase = h * HID

            def _dot(i2, accs):
                new = []
                for t in range(4):
                    dcol = hbase + ((iota16 + (i2 * 4 + t)) & (HID - 1))
                    qv = plsc.load_gather(qb, [lanes, dcol])
                    kv = plsc.load_gather(kb, [lanes, dcol])
                    new.append(accs[t] + qv * kv)
                return tuple(new)
            a4 = plsc.parallel_loop(
                0, HID // 4, 1, unroll=2,
                carry=(zeros16, zeros16, zeros16, zeros16))(_dot)
            a = (a4[0] + a4[1]) + (a4[2] + a4[3])
            exv = jnp.exp(a * INV_SQRT_HID)
            plsc.store_scatter(
                qb, [lanes, jnp.full((L,), DQK, jnp.int32) + h], exv)

            def _scale(i2):
                dcol = hbase + ((iota16 + i2) & (HID - 1))
                qv = plsc.load_gather(qb, [lanes, dcol])
                plsc.store_scatter(qb, [lanes, dcol], qv * exv)
            plsc.parallel_loop(0, HID, 1, unroll=4)(_scale)
            return carry
        lax.fori_loop(0, (CHUNK // L) * H, _gh, 0)

    sem_gs = (sem_g0, sem_g1)

    def _iter(j2, carry):
        m = 2 * j2
        for a in range(2):
            cur_row, cur_col, cur_sem = blks[a]
            nxt_row, nxt_col, nxt_sem = blks[1 - a]
            for t in range(IB):
                b = t % 2
                o = 1 - b
                if t == IB - 1:
                    pltpu.make_async_copy(row_hbm.at[pl.ds(cbase, IB)], nxt_row, nxt_sem).wait()
                    pltpu.make_async_copy(col_hbm.at[pl.ds(cbase, IB)], nxt_col, nxt_sem).wait()
                    gr, gc = nxt_row.at[0], nxt_col.at[0]
                else:
                    gr, gc = cur_row.at[t + 1], cur_col.at[t + 1]
                pltpu.async_copy(q_hbm.at[gr], qbs[o], sem_gs[o])
                pltpu.async_copy(k_hbm.at[gc], kbs[o], sem_gs[o])
                pltpu.make_async_copy(q_hbm.at[gr], qbs[b], sem_gs[b]).wait()
                pltpu.make_async_copy(k_hbm.at[gc], kbs[b], sem_gs[b]).wait()
                _compute(qbs[b], kbs[b])
                pltpu.sync_copy(qbs[b], acc_sh.at[cur_col.at[t]], add=True)
            nb = jnp.minimum(m + a + 2, NBLK - 1)
            pltpu.async_copy(row_hbm.at[pl.ds(cbase + nb * IB, IB)], cur_row, cur_sem)
            pltpu.async_copy(col_hbm.at[pl.ds(cbase + nb * IB, IB)], cur_col, cur_sem)
        return carry
    lax.fori_loop(0, NBLK // 2, _iter, 0)

    pltpu.make_async_copy(q_hbm.at[rowA.at[0]], qb0, sem_g0).wait()
    pltpu.make_async_copy(k_hbm.at[colA.at[0]], kb0, sem_g0).wait()
    pltpu.make_async_copy(row_hbm.at[pl.ds(cbase, IB)], rowB, sem_iB).wait()
    pltpu.make_async_copy(col_hbm.at[pl.ds(cbase, IB)], colB, sem_iB).wait()

    plsc.subcore_barrier()

    for t in range(ROWS_PER_TILE // CHUNK):
        pltpu.sync_copy(acc_sh.at[pl.ds(base_r + t * CHUNK, CHUNK)],
                        out_hbm.at[c, pl.ds(base_r + t * CHUNK, CHUNK)])
    if rem:
        pltpu.sync_copy(acc_sh.at[pl.ds(base_r + (ROWS_PER_TILE // CHUNK) * CHUNK, rem)],
                        out_hbm.at[c, pl.ds(base_r + (ROWS_PER_TILE // CHUNK) * CHUNK, rem)])


_edge_call = functools.partial(
    pl.kernel,
    out_type=jax.ShapeDtypeStruct((NC, NPAD, WACC), jnp.float32),
    mesh=plsc.VectorSubcoreMesh(core_axis_name="c", subcore_axis_name="s"),
    compiler_params=pltpu.CompilerParams(use_tc_tiling_on_sc=False,
                                         needs_layout_passes=False),
    scratch_types=[
        pltpu.VMEM((IB, CHUNK), jnp.int32),
        pltpu.VMEM((IB, CHUNK), jnp.int32),
        pltpu.VMEM((IB, CHUNK), jnp.int32),
        pltpu.VMEM((IB, CHUNK), jnp.int32),
        pltpu.VMEM((CHUNK, WACC), jnp.float32),
        pltpu.VMEM((CHUNK, DQK), jnp.float32),
        pltpu.VMEM((CHUNK, WACC), jnp.float32),
        pltpu.VMEM((CHUNK, DQK), jnp.float32),
        pltpu.VMEM_SHARED((NPAD, WACC), jnp.float32),
        pltpu.SemaphoreType.DMA,
        pltpu.SemaphoreType.DMA,
        pltpu.SemaphoreType.DMA,
        pltpu.SemaphoreType.DMA,
    ],
)(_edge_body)


BLKF = 2000


def _fin_body(acc_ref, g_ref, b_ref, wo_ref, bo_ref, o_ref):
    a = acc_ref[0] + acc_ref[1]
    msg = a[:, :DQK]
    sums = a[:, DQK:DQK + H]
    recip = 1.0 / (sums + 1e-16)
    hh = lax.broadcasted_iota(jnp.int32, (H, DQK), 0)
    dd = lax.broadcasted_iota(jnp.int32, (H, DQK), 1) // HID
    proj = (hh == dd).astype(jnp.float32)
    bc = lax.dot_general(recip, proj, (((1,), (0,)), ((), ())),
                         preferred_element_type=jnp.float32)
    xc = msg * bc
    mu = jnp.mean(xc, axis=1, keepdims=True)
    var = jnp.mean((xc - mu) ** 2, axis=1, keepdims=True)
    xn = (xc - mu) / jnp.sqrt(var + 1e-5) * g_ref[...] + b_ref[...]
    xs = xn * jax.nn.sigmoid(xn)
    o_ref[...] = lax.dot_general(xs, wo_ref[...], (((1,), (1,)), ((), ())),
                                 preferred_element_type=jnp.float32) + bo_ref[...]


def _finish(acc, g2, b2, wo, bo2):
    return pl.pallas_call(
        _fin_body,
        grid=(N // BLKF,),
        in_specs=[
            pl.BlockSpec((NC, BLKF, WACC), lambda i: (0, i, 0)),
            pl.BlockSpec((1, DQK), lambda i: (0, 0)),
            pl.BlockSpec((1, DQK), lambda i: (0, 0)),
            pl.BlockSpec((OUT, DQK), lambda i: (0, 0)),
            pl.BlockSpec((1, OUT), lambda i: (0, 0)),
        ],
        out_specs=pl.BlockSpec((BLKF, OUT), lambda i: (i, 0)),
        out_shape=jax.ShapeDtypeStruct((N, OUT), jnp.float32),
    )(acc, g2, b2, wo, bo2)


def kernel(x, edge_index, Wq, Wk, ln_gamma, ln_beta, Wo, bo):
    wq2 = Wq.reshape(DQK, IN)
    wk2 = Wk.reshape(DQK, IN)
    q, k = _project(x, wq2, wk2)

    pad = EPAD - E
    rowp = jnp.concatenate([edge_index[0], jnp.zeros((pad,), jnp.int32)])
    colp = jnp.concatenate([edge_index[1], jnp.full((pad,), N, jnp.int32)])
    rowp = rowp.reshape(EPAD // CHUNK, CHUNK)
    colp = colp.reshape(EPAD // CHUNK, CHUNK)
    acc = _edge_call(q, k, rowp, colp)

    return _finish(acc, ln_gamma.reshape(1, DQK), ln_beta.reshape(1, DQK),
                   Wo, bo.reshape(1, OUT))
